# baseline (speedup 1.0000x reference)
import functools

import jax
import jax.numpy as jnp
from jax import lax
from jax.experimental import pallas as pl
from jax.experimental.pallas import tpu as pltpu
from jax.experimental.pallas import tpu_sc as plsc

_K = 8192
_D = 256
_N = 9216
_TB = 768
_KB = 1024
_NT = _N // _TB
_NK = _K // _KB
_CC = 0.25

_NW = 32
_BPW = _N // _NW
_CH = 96
_NCH = _BPW // _CH


def _dist_body(x_ref, w_ref, idx_ref, minv_ref, psum_ref, w2_ref, ws_ref):
    i = pl.program_id(0)
    xt = jnp.swapaxes(x_ref[...], 0, 1)
    x2 = jnp.sum(xt * xt, axis=0, keepdims=True)

    @pl.when(i == 0)
    def _():
        for k in range(_NK):
            w = w_ref[pl.ds(k * _KB, _KB), :]
            w2_ref[pl.ds(k * _KB, _KB), :] = jnp.sum(w * w, axis=1,
                                                     keepdims=True)
            ws_ref[pl.ds(k * _KB, _KB), :] = w * -2.0
        psum_ref[0, 0] = 0.0

    best = None
    bidx = None
    for k in range(_NK):
        ws = ws_ref[pl.ds(k * _KB, _KB), :]
        mm2 = lax.dot_general(ws, xt, (((1,), (0,)), ((), ())),
                              preferred_element_type=jnp.float32)
        w2 = w2_ref[pl.ds(k * _KB, _KB), :]
        d = (x2 + w2) + mm2
        m = jnp.min(d, axis=0, keepdims=True)
        iota = lax.broadcasted_iota(jnp.int32, (_KB, _TB), 0)
        li = jnp.min(jnp.where(d == m, iota, _K), axis=0, keepdims=True)
        gi = li + k * _KB
        if k == 0:
            best, bidx = m, gi
        else:
            upd = m < best
            best = jnp.where(upd, m, best)
            bidx = jnp.where(upd, gi, bidx)
    idx_ref[0] = bidx
    minv_ref[0] = best
    psum_ref[0, 0] += jnp.sum(best)


def _make_dist_call(ntok):
    nt = ntok // _TB
    return pl.pallas_call(
        _dist_body,
        grid=(nt,),
        in_specs=[
            pl.BlockSpec((_TB, _D), lambda i: (i, 0)),
            pl.BlockSpec((_K, _D), lambda i: (0, 0)),
        ],
        out_specs=[
            pl.BlockSpec((1, 1, _TB), lambda i: (i, 0, 0)),
            pl.BlockSpec((1, 1, _TB), lambda i: (i, 0, 0)),
            pl.BlockSpec((1, 1), lambda i: (0, 0), memory_space=pltpu.SMEM),
        ],
        out_shape=[
            jax.ShapeDtypeStruct((nt, 1, _TB), jnp.int32),
            jax.ShapeDtypeStruct((nt, 1, _TB), jnp.float32),
            jax.ShapeDtypeStruct((1, 1), jnp.float32),
        ],
        scratch_shapes=[
            pltpu.VMEM((_K, 1), jnp.float32),
            pltpu.VMEM((_K, _D), jnp.float32),
        ],
        compiler_params=pltpu.CompilerParams(
            dimension_semantics=("arbitrary",),
        ),
    )


@functools.cache
def _make_gather(ntok):
    bpw = ntok // _NW
    nch = bpw // _CH

    @functools.partial(
        pl.kernel,
        mesh=plsc.VectorSubcoreMesh(core_axis_name="c", subcore_axis_name="s"),
        out_type=jax.ShapeDtypeStruct((ntok, _D), jnp.float32),
        scratch_types=[
            pltpu.VMEM((nch, _CH), jnp.int32),
            pltpu.VMEM((nch, _CH, _D), jnp.float32),
            pltpu.SemaphoreType.DMA,
        ],
    )
    def _gather(w_hbm, idx_hbm, out_hbm, idx_v, rows_v, sem):
        wid = lax.axis_index("s") * 2 + lax.axis_index("c")
        base = wid * bpw
        copies = []
        for j in range(nch):
            pltpu.sync_copy(idx_hbm.at[pl.ds(base + j * _CH, _CH)], idx_v.at[j])
            copies.append(
                pltpu.async_copy(w_hbm.at[idx_v.at[j]], rows_v.at[j], sem))
        for j in range(nch):
            copies[j].wait()
            pltpu.sync_copy(rows_v.at[j], out_hbm.at[pl.ds(base + j * _CH, _CH)])

    return _gather


def kernel(inputs, topic_embedding, theta, pretrain_vq, W):
    del theta
    Wsel = jnp.where(pretrain_vq != 0, topic_embedding, W)
    flat = inputs.reshape(_N, _D)
    half = _N // 2
    dist = _make_dist_call(half)
    gather = _make_gather(half)
    idx_h, minv_h, psum_h, q_h = [], [], [], []
    for h in range(2):
        idx2, minv2, psums = dist(flat[h * half:(h + 1) * half], Wsel)
        idx_h.append(idx2.reshape(half))
        minv_h.append(minv2.reshape(half))
        psum_h.append(psums[0, 0])
        q_h.append(gather(Wsel, idx_h[h]))
    idx = jnp.concatenate(idx_h)
    minv = jnp.concatenate(minv_h)
    quantized = jnp.concatenate(q_h)
    loss_commit = psum_h[0] + psum_h[1]
    loss = loss_commit * _CC + loss_commit
    quantized_st = quantized.reshape(inputs.shape)
    encoding_indices = idx.reshape(inputs.shape[:-1])
    min_distances = minv.reshape(inputs.shape[:-1])
    return (quantized_st, loss, encoding_indices, min_distances, loss_commit)


# speedup vs baseline: 1.1564x; 1.1564x over previous
import functools

import jax
import jax.numpy as jnp
from jax import lax
from jax.experimental import pallas as pl
from jax.experimental.pallas import tpu as pltpu
from jax.experimental.pallas import tpu_sc as plsc

_K = 8192
_D = 256
_N = 9216
_TB = 768
_KB = 1024
_NT = _N // _TB
_NK = _K // _KB
_CC = 0.25

_NW = 32
_BPW = _N // _NW
_CH = 96
_NCH = _BPW // _CH


def _dist_body(x_ref, w_ref, idx_ref, minv_ref, psum_ref, w2_ref, ws_ref):
    i = pl.program_id(0)
    xt = jnp.swapaxes(x_ref[...], 0, 1)
    x2 = jnp.sum(xt * xt, axis=0, keepdims=True)

    @pl.when(i == 0)
    def _():
        for k in range(_NK):
            w = w_ref[pl.ds(k * _KB, _KB), :]
            w2_ref[pl.ds(k * _KB, _KB), :] = jnp.sum(w * w, axis=1,
                                                     keepdims=True)
            ws_ref[pl.ds(k * _KB, _KB), :] = w * -2.0
        psum_ref[0, 0] = 0.0

    best = None
    bidx = None
    for k in range(_NK):
        ws = ws_ref[pl.ds(k * _KB, _KB), :]
        mm2 = lax.dot_general(ws, xt, (((1,), (0,)), ((), ())),
                              preferred_element_type=jnp.float32)
        w2 = w2_ref[pl.ds(k * _KB, _KB), :]
        d = (x2 + w2) + mm2
        m = jnp.min(d, axis=0, keepdims=True)
        iota = lax.broadcasted_iota(jnp.int32, (_KB, _TB), 0)
        li = jnp.min(jnp.where(d == m, iota, _K), axis=0, keepdims=True)
        gi = li + k * _KB
        if k == 0:
            best, bidx = m, gi
        else:
            upd = m < best
            best = jnp.where(upd, m, best)
            bidx = jnp.where(upd, gi, bidx)
    idx_ref[0] = bidx
    minv_ref[0] = best
    psum_ref[0, 0] += jnp.sum(best)


def _make_dist_call(ntok):
    nt = ntok // _TB
    return pl.pallas_call(
        _dist_body,
        grid=(nt,),
        in_specs=[
            pl.BlockSpec((_TB, _D), lambda i: (i, 0)),
            pl.BlockSpec((_K, _D), lambda i: (0, 0)),
        ],
        out_specs=[
            pl.BlockSpec((1, 1, _TB), lambda i: (i, 0, 0)),
            pl.BlockSpec((1, 1, _TB), lambda i: (i, 0, 0)),
            pl.BlockSpec((1, 1), lambda i: (0, 0), memory_space=pltpu.SMEM),
        ],
        out_shape=[
            jax.ShapeDtypeStruct((nt, 1, _TB), jnp.int32),
            jax.ShapeDtypeStruct((nt, 1, _TB), jnp.float32),
            jax.ShapeDtypeStruct((1, 1), jnp.float32),
        ],
        scratch_shapes=[
            pltpu.VMEM((_K, 1), jnp.float32),
            pltpu.VMEM((_K, _D), jnp.float32),
        ],
        compiler_params=pltpu.CompilerParams(
            dimension_semantics=("arbitrary",),
        ),
    )


@functools.cache
def _make_gather(ntok):
    bpw = ntok // _NW
    nch = bpw // _CH

    @functools.partial(
        pl.kernel,
        mesh=plsc.VectorSubcoreMesh(core_axis_name="c", subcore_axis_name="s"),
        out_type=jax.ShapeDtypeStruct((ntok, _D), jnp.float32),
        scratch_types=[
            pltpu.VMEM((nch, _CH), jnp.int32),
            pltpu.VMEM((nch, _CH, _D), jnp.float32),
            pltpu.SemaphoreType.DMA,
        ],
    )
    def _gather(w_hbm, idx_hbm, out_hbm, idx_v, rows_v, sem):
        wid = lax.axis_index("s") * 2 + lax.axis_index("c")
        base = wid * bpw
        copies = []
        for j in range(nch):
            pltpu.sync_copy(idx_hbm.at[pl.ds(base + j * _CH, _CH)], idx_v.at[j])
            copies.append(
                pltpu.async_copy(w_hbm.at[idx_v.at[j]], rows_v.at[j], sem))
        for j in range(nch):
            copies[j].wait()
            pltpu.sync_copy(rows_v.at[j], out_hbm.at[pl.ds(base + j * _CH, _CH)])

    return _gather


def kernel(inputs, topic_embedding, theta, pretrain_vq, W):
    del theta
    Wsel = jnp.where(pretrain_vq != 0, topic_embedding, W)
    flat = inputs.reshape(_N, _D)
    idx2, minv2, psums = _make_dist_call(_N)(flat, Wsel)
    idx = idx2.reshape(_N)
    quantized = _make_gather(_N)(Wsel, idx)
    loss_commit = psums[0, 0]
    loss = loss_commit * _CC + loss_commit
    quantized_st = quantized.reshape(inputs.shape)
    encoding_indices = idx.reshape(inputs.shape[:-1])
    min_distances = minv2.reshape(inputs.shape[:-1])
    return (quantized_st, loss, encoding_indices, min_distances, loss_commit)


# speedup vs baseline: 1.2354x; 1.0683x over previous
import functools

import jax
import jax.numpy as jnp
from jax import lax
from jax.experimental import pallas as pl
from jax.experimental.pallas import tpu as pltpu
from jax.experimental.pallas import tpu_sc as plsc

_K = 8192
_D = 256
_N = 9216
_TB = 768
_KB = 1024
_NT = _N // _TB
_NK = _K // _KB
_CC = 0.25

_NW = 32
_BPW = _N // _NW
_CH = 96
_NCH = _BPW // _CH


def _dist_body(x_ref, w_ref, idx_ref, minv_ref, psum_ref, w2_ref, ws_ref):
    i = pl.program_id(0)
    xt = jnp.swapaxes(x_ref[...], 0, 1)
    x2 = jnp.sum(xt * xt, axis=0, keepdims=True)

    @pl.when(i == 0)
    def _():
        for k in range(_NK):
            w = w_ref[pl.ds(k * _KB, _KB), :]
            w2_ref[pl.ds(k * _KB, _KB), :] = jnp.sum(w * w, axis=1,
                                                     keepdims=True)
            ws_ref[pl.ds(k * _KB, _KB), :] = w * -2.0
        psum_ref[0, 0] = 0.0

    iota = lax.broadcasted_iota(jnp.int32, (_KB, _TB), 0).astype(jnp.float32)
    best = None
    bidx = None
    for k in range(_NK):
        ws = ws_ref[pl.ds(k * _KB, _KB), :]
        mm2 = lax.dot_general(ws, xt, (((1,), (0,)), ((), ())),
                              preferred_element_type=jnp.float32)
        w2 = w2_ref[pl.ds(k * _KB, _KB), :]
        d = (x2 + w2) + mm2
        m = jnp.min(d, axis=0, keepdims=True)
        li = jnp.min(jnp.where(d == m, iota, float(_K)), axis=0,
                     keepdims=True)
        gi = li + float(k * _KB)
        if k == 0:
            best, bidx = m, gi
        else:
            upd = m < best
            best = jnp.where(upd, m, best)
            bidx = jnp.where(upd, gi, bidx)
    idx_ref[0] = bidx.astype(jnp.int32)
    minv_ref[0] = best
    psum_ref[0, 0] += jnp.sum(best)


def _make_dist_call(ntok):
    nt = ntok // _TB
    return pl.pallas_call(
        _dist_body,
        grid=(nt,),
        in_specs=[
            pl.BlockSpec((_TB, _D), lambda i: (i, 0)),
            pl.BlockSpec((_K, _D), lambda i: (0, 0)),
        ],
        out_specs=[
            pl.BlockSpec((1, 1, _TB), lambda i: (i, 0, 0)),
            pl.BlockSpec((1, 1, _TB), lambda i: (i, 0, 0)),
            pl.BlockSpec((1, 1), lambda i: (0, 0), memory_space=pltpu.SMEM),
        ],
        out_shape=[
            jax.ShapeDtypeStruct((nt, 1, _TB), jnp.int32),
            jax.ShapeDtypeStruct((nt, 1, _TB), jnp.float32),
            jax.ShapeDtypeStruct((1, 1), jnp.float32),
        ],
        scratch_shapes=[
            pltpu.VMEM((_K, 1), jnp.float32),
            pltpu.VMEM((_K, _D), jnp.float32),
        ],
        compiler_params=pltpu.CompilerParams(
            dimension_semantics=("arbitrary",),
        ),
    )


@functools.cache
def _make_gather(ntok):
    bpw = ntok // _NW
    nch = bpw // _CH

    @functools.partial(
        pl.kernel,
        mesh=plsc.VectorSubcoreMesh(core_axis_name="c", subcore_axis_name="s"),
        out_type=jax.ShapeDtypeStruct((ntok, _D), jnp.float32),
        scratch_types=[
            pltpu.VMEM((nch, _CH), jnp.int32),
            pltpu.VMEM((nch, _CH, _D), jnp.float32),
            pltpu.SemaphoreType.DMA,
        ],
    )
    def _gather(w_hbm, idx_hbm, out_hbm, idx_v, rows_v, sem):
        wid = lax.axis_index("s") * 2 + lax.axis_index("c")
        base = wid * bpw
        copies = []
        for j in range(nch):
            pltpu.sync_copy(idx_hbm.at[pl.ds(base + j * _CH, _CH)], idx_v.at[j])
            copies.append(
                pltpu.async_copy(w_hbm.at[idx_v.at[j]], rows_v.at[j], sem))
        for j in range(nch):
            copies[j].wait()
            pltpu.sync_copy(rows_v.at[j], out_hbm.at[pl.ds(base + j * _CH, _CH)])

    return _gather


def kernel(inputs, topic_embedding, theta, pretrain_vq, W):
    del theta
    Wsel = jnp.where(pretrain_vq != 0, topic_embedding, W)
    flat = inputs.reshape(_N, _D)
    idx2, minv2, psums = _make_dist_call(_N)(flat, Wsel)
    idx = idx2.reshape(_N)
    quantized = _make_gather(_N)(Wsel, idx)
    loss_commit = psums[0, 0]
    loss = loss_commit * _CC + loss_commit
    quantized_st = quantized.reshape(inputs.shape)
    encoding_indices = idx.reshape(inputs.shape[:-1])
    min_distances = minv2.reshape(inputs.shape[:-1])
    return (quantized_st, loss, encoding_indices, min_distances, loss_commit)


# speedup vs baseline: 1.2956x; 1.0487x over previous
import functools

import jax
import jax.numpy as jnp
from jax import lax
from jax.experimental import pallas as pl
from jax.experimental.pallas import tpu as pltpu
from jax.experimental.pallas import tpu_sc as plsc

_K = 8192
_D = 256
_N = 9216
_TB = 1536
_KB = 2048
_NT = _N // _TB
_NK = _K // _KB
_CC = 0.25

_NW = 32
_BPW = _N // _NW
_CH = 96
_NCH = _BPW // _CH


def _dist_body(x_ref, w_ref, idx_ref, minv_ref, psum_ref, w2_ref, ws_ref):
    i = pl.program_id(0)
    xt = jnp.swapaxes(x_ref[...], 0, 1)
    x2 = jnp.sum(xt * xt, axis=0, keepdims=True)

    @pl.when(i == 0)
    def _():
        for k in range(_NK):
            w = w_ref[pl.ds(k * _KB, _KB), :]
            w2_ref[pl.ds(k * _KB, _KB), :] = jnp.sum(w * w, axis=1,
                                                     keepdims=True)
            ws_ref[pl.ds(k * _KB, _KB), :] = w * -2.0
        psum_ref[0, 0] = 0.0

    iota = lax.broadcasted_iota(jnp.int32, (_KB, _TB), 0).astype(jnp.float32)
    best = None
    bidx = None
    for k in range(_NK):
        ws = ws_ref[pl.ds(k * _KB, _KB), :]
        mm2 = lax.dot_general(ws, xt, (((1,), (0,)), ((), ())),
                              preferred_element_type=jnp.float32)
        w2 = w2_ref[pl.ds(k * _KB, _KB), :]
        d = (x2 + w2) + mm2
        m = jnp.min(d, axis=0, keepdims=True)
        li = jnp.min(jnp.where(d == m, iota, float(_K)), axis=0,
                     keepdims=True)
        gi = li + float(k * _KB)
        if k == 0:
            best, bidx = m, gi
        else:
            upd = m < best
            best = jnp.where(upd, m, best)
            bidx = jnp.where(upd, gi, bidx)
    idx_ref[0] = bidx.astype(jnp.int32)
    minv_ref[0] = best
    psum_ref[0, 0] += jnp.sum(best)


def _make_dist_call(ntok):
    nt = ntok // _TB
    return pl.pallas_call(
        _dist_body,
        grid=(nt,),
        in_specs=[
            pl.BlockSpec((_TB, _D), lambda i: (i, 0)),
            pl.BlockSpec((_K, _D), lambda i: (0, 0)),
        ],
        out_specs=[
            pl.BlockSpec((1, 1, _TB), lambda i: (i, 0, 0)),
            pl.BlockSpec((1, 1, _TB), lambda i: (i, 0, 0)),
            pl.BlockSpec((1, 1), lambda i: (0, 0), memory_space=pltpu.SMEM),
        ],
        out_shape=[
            jax.ShapeDtypeStruct((nt, 1, _TB), jnp.int32),
            jax.ShapeDtypeStruct((nt, 1, _TB), jnp.float32),
            jax.ShapeDtypeStruct((1, 1), jnp.float32),
        ],
        scratch_shapes=[
            pltpu.VMEM((_K, 1), jnp.float32),
            pltpu.VMEM((_K, _D), jnp.float32),
        ],
        compiler_params=pltpu.CompilerParams(
            dimension_semantics=("arbitrary",),
        ),
    )


@functools.cache
def _make_gather(ntok):
    bpw = ntok // _NW
    nch = bpw // _CH

    @functools.partial(
        pl.kernel,
        mesh=plsc.VectorSubcoreMesh(core_axis_name="c", subcore_axis_name="s"),
        out_type=jax.ShapeDtypeStruct((ntok, _D), jnp.float32),
        scratch_types=[
            pltpu.VMEM((nch, _CH), jnp.int32),
            pltpu.VMEM((nch, _CH, _D), jnp.float32),
            pltpu.SemaphoreType.DMA,
        ],
    )
    def _gather(w_hbm, idx_hbm, out_hbm, idx_v, rows_v, sem):
        wid = lax.axis_index("s") * 2 + lax.axis_index("c")
        base = wid * bpw
        copies = []
        for j in range(nch):
            pltpu.sync_copy(idx_hbm.at[pl.ds(base + j * _CH, _CH)], idx_v.at[j])
            copies.append(
                pltpu.async_copy(w_hbm.at[idx_v.at[j]], rows_v.at[j], sem))
        for j in range(nch):
            copies[j].wait()
            pltpu.sync_copy(rows_v.at[j], out_hbm.at[pl.ds(base + j * _CH, _CH)])

    return _gather


def kernel(inputs, topic_embedding, theta, pretrain_vq, W):
    del theta
    Wsel = jnp.where(pretrain_vq != 0, topic_embedding, W)
    flat = inputs.reshape(_N, _D)
    idx2, minv2, psums = _make_dist_call(_N)(flat, Wsel)
    idx = idx2.reshape(_N)
    quantized = _make_gather(_N)(Wsel, idx)
    loss_commit = psums[0, 0]
    loss = loss_commit * _CC + loss_commit
    quantized_st = quantized.reshape(inputs.shape)
    encoding_indices = idx.reshape(inputs.shape[:-1])
    min_distances = minv2.reshape(inputs.shape[:-1])
    return (quantized_st, loss, encoding_indices, min_distances, loss_commit)


# speedup vs baseline: 1.3038x; 1.0063x over previous
import functools

import jax
import jax.numpy as jnp
from jax import lax
from jax.experimental import pallas as pl
from jax.experimental.pallas import tpu as pltpu
from jax.experimental.pallas import tpu_sc as plsc

_K = 8192
_D = 256
_N = 9216
_TB = 2304
_KB = 1024
_NT = _N // _TB
_NK = _K // _KB
_CC = 0.25

_NW = 32
_BPW = _N // _NW
_CH = 96
_NCH = _BPW // _CH


def _dist_body(x_ref, w_ref, idx_ref, minv_ref, psum_ref, w2_ref, ws_ref):
    i = pl.program_id(0)
    xt = jnp.swapaxes(x_ref[...], 0, 1)
    x2 = jnp.sum(xt * xt, axis=0, keepdims=True)

    @pl.when(i == 0)
    def _():
        for k in range(_NK):
            w = w_ref[pl.ds(k * _KB, _KB), :]
            w2_ref[pl.ds(k * _KB, _KB), :] = jnp.sum(w * w, axis=1,
                                                     keepdims=True)
            ws_ref[pl.ds(k * _KB, _KB), :] = w * -2.0
        psum_ref[0, 0] = 0.0

    iota = lax.broadcasted_iota(jnp.int32, (_KB, _TB), 0).astype(jnp.float32)
    best = None
    bidx = None
    for k in range(_NK):
        ws = ws_ref[pl.ds(k * _KB, _KB), :]
        mm2 = lax.dot_general(ws, xt, (((1,), (0,)), ((), ())),
                              preferred_element_type=jnp.float32)
        w2 = w2_ref[pl.ds(k * _KB, _KB), :]
        d = (x2 + w2) + mm2
        m = jnp.min(d, axis=0, keepdims=True)
        li = jnp.min(jnp.where(d == m, iota, float(_K)), axis=0,
                     keepdims=True)
        gi = li + float(k * _KB)
        if k == 0:
            best, bidx = m, gi
        else:
            upd = m < best
            best = jnp.where(upd, m, best)
            bidx = jnp.where(upd, gi, bidx)
    idx_ref[0] = bidx.astype(jnp.int32)
    minv_ref[0] = best
    psum_ref[0, 0] += jnp.sum(best)


def _make_dist_call(ntok):
    nt = ntok // _TB
    return pl.pallas_call(
        _dist_body,
        grid=(nt,),
        in_specs=[
            pl.BlockSpec((_TB, _D), lambda i: (i, 0)),
            pl.BlockSpec((_K, _D), lambda i: (0, 0)),
        ],
        out_specs=[
            pl.BlockSpec((1, 1, _TB), lambda i: (i, 0, 0)),
            pl.BlockSpec((1, 1, _TB), lambda i: (i, 0, 0)),
            pl.BlockSpec((1, 1), lambda i: (0, 0), memory_space=pltpu.SMEM),
        ],
        out_shape=[
            jax.ShapeDtypeStruct((nt, 1, _TB), jnp.int32),
            jax.ShapeDtypeStruct((nt, 1, _TB), jnp.float32),
            jax.ShapeDtypeStruct((1, 1), jnp.float32),
        ],
        scratch_shapes=[
            pltpu.VMEM((_K, 1), jnp.float32),
            pltpu.VMEM((_K, _D), jnp.float32),
        ],
        compiler_params=pltpu.CompilerParams(
            dimension_semantics=("arbitrary",),
        ),
    )


@functools.cache
def _make_gather(ntok):
    bpw = ntok // _NW
    nch = bpw // _CH

    @functools.partial(
        pl.kernel,
        mesh=plsc.VectorSubcoreMesh(core_axis_name="c", subcore_axis_name="s"),
        out_type=jax.ShapeDtypeStruct((ntok, _D), jnp.float32),
        scratch_types=[
            pltpu.VMEM((nch, _CH), jnp.int32),
            pltpu.VMEM((nch, _CH, _D), jnp.float32),
            pltpu.SemaphoreType.DMA,
        ],
    )
    def _gather(w_hbm, idx_hbm, out_hbm, idx_v, rows_v, sem):
        wid = lax.axis_index("s") * 2 + lax.axis_index("c")
        base = wid * bpw
        copies = []
        for j in range(nch):
            pltpu.sync_copy(idx_hbm.at[pl.ds(base + j * _CH, _CH)], idx_v.at[j])
            copies.append(
                pltpu.async_copy(w_hbm.at[idx_v.at[j]], rows_v.at[j], sem))
        for j in range(nch):
            copies[j].wait()
            pltpu.sync_copy(rows_v.at[j], out_hbm.at[pl.ds(base + j * _CH, _CH)])

    return _gather


def kernel(inputs, topic_embedding, theta, pretrain_vq, W):
    del theta
    Wsel = jnp.where(pretrain_vq != 0, topic_embedding, W)
    flat = inputs.reshape(_N, _D)
    idx2, minv2, psums = _make_dist_call(_N)(flat, Wsel)
    idx = idx2.reshape(_N)
    quantized = _make_gather(_N)(Wsel, idx)
    loss_commit = psums[0, 0]
    loss = loss_commit * _CC + loss_commit
    quantized_st = quantized.reshape(inputs.shape)
    encoding_indices = idx.reshape(inputs.shape[:-1])
    min_distances = minv2.reshape(inputs.shape[:-1])
    return (quantized_st, loss, encoding_indices, min_distances, loss_commit)


# speedup vs baseline: 1.3156x; 1.0090x over previous
import functools

import jax
import jax.numpy as jnp
from jax import lax
from jax.experimental import pallas as pl
from jax.experimental.pallas import tpu as pltpu
from jax.experimental.pallas import tpu_sc as plsc

_K = 8192
_D = 256
_N = 9216
_TB = 2304
_KB = 2048
_NT = _N // _TB
_NK = _K // _KB
_CC = 0.25

_NW = 32
_BPW = _N // _NW
_CH = 96
_NCH = _BPW // _CH


def _dist_body(x_ref, w_ref, idx_ref, minv_ref, psum_ref, w2_ref, ws_ref):
    i = pl.program_id(0)
    xt = jnp.swapaxes(x_ref[...], 0, 1)
    x2 = jnp.sum(xt * xt, axis=0, keepdims=True)

    @pl.when(i == 0)
    def _():
        for k in range(_NK):
            w = w_ref[pl.ds(k * _KB, _KB), :]
            w2_ref[pl.ds(k * _KB, _KB), :] = jnp.sum(w * w, axis=1,
                                                     keepdims=True)
            ws_ref[pl.ds(k * _KB, _KB), :] = w * -2.0
        psum_ref[0, 0] = 0.0

    iota = lax.broadcasted_iota(jnp.int32, (_KB, _TB), 0).astype(jnp.float32)
    best = None
    bidx = None
    for k in range(_NK):
        ws = ws_ref[pl.ds(k * _KB, _KB), :]
        mm2 = lax.dot_general(ws, xt, (((1,), (0,)), ((), ())),
                              preferred_element_type=jnp.float32)
        w2 = w2_ref[pl.ds(k * _KB, _KB), :]
        d = (x2 + w2) + mm2
        m = jnp.min(d, axis=0, keepdims=True)
        li = jnp.min(jnp.where(d == m, iota, float(_K)), axis=0,
                     keepdims=True)
        gi = li + float(k * _KB)
        if k == 0:
            best, bidx = m, gi
        else:
            upd = m < best
            best = jnp.where(upd, m, best)
            bidx = jnp.where(upd, gi, bidx)
    idx_ref[0] = bidx.astype(jnp.int32)
    minv_ref[0] = best
    psum_ref[0, 0] += jnp.sum(best)


def _make_dist_call(ntok):
    nt = ntok // _TB
    return pl.pallas_call(
        _dist_body,
        grid=(nt,),
        in_specs=[
            pl.BlockSpec((_TB, _D), lambda i: (i, 0)),
            pl.BlockSpec((_K, _D), lambda i: (0, 0)),
        ],
        out_specs=[
            pl.BlockSpec((1, 1, _TB), lambda i: (i, 0, 0)),
            pl.BlockSpec((1, 1, _TB), lambda i: (i, 0, 0)),
            pl.BlockSpec((1, 1), lambda i: (0, 0), memory_space=pltpu.SMEM),
        ],
        out_shape=[
            jax.ShapeDtypeStruct((nt, 1, _TB), jnp.int32),
            jax.ShapeDtypeStruct((nt, 1, _TB), jnp.float32),
            jax.ShapeDtypeStruct((1, 1), jnp.float32),
        ],
        scratch_shapes=[
            pltpu.VMEM((_K, 1), jnp.float32),
            pltpu.VMEM((_K, _D), jnp.float32),
        ],
        compiler_params=pltpu.CompilerParams(
            dimension_semantics=("arbitrary",),
        ),
    )


@functools.cache
def _make_gather(ntok):
    bpw = ntok // _NW
    nch = bpw // _CH

    @functools.partial(
        pl.kernel,
        mesh=plsc.VectorSubcoreMesh(core_axis_name="c", subcore_axis_name="s"),
        out_type=jax.ShapeDtypeStruct((ntok, _D), jnp.float32),
        scratch_types=[
            pltpu.VMEM((nch, _CH), jnp.int32),
            pltpu.VMEM((nch, _CH, _D), jnp.float32),
            pltpu.SemaphoreType.DMA,
        ],
    )
    def _gather(w_hbm, idx_hbm, out_hbm, idx_v, rows_v, sem):
        wid = lax.axis_index("s") * 2 + lax.axis_index("c")
        base = wid * bpw
        copies = []
        for j in range(nch):
            pltpu.sync_copy(idx_hbm.at[pl.ds(base + j * _CH, _CH)], idx_v.at[j])
            copies.append(
                pltpu.async_copy(w_hbm.at[idx_v.at[j]], rows_v.at[j], sem))
        for j in range(nch):
            copies[j].wait()
            pltpu.sync_copy(rows_v.at[j], out_hbm.at[pl.ds(base + j * _CH, _CH)])

    return _gather


def kernel(inputs, topic_embedding, theta, pretrain_vq, W):
    del theta
    Wsel = jnp.where(pretrain_vq != 0, topic_embedding, W)
    flat = inputs.reshape(_N, _D)
    idx2, minv2, psums = _make_dist_call(_N)(flat, Wsel)
    idx = idx2.reshape(_N)
    quantized = _make_gather(_N)(Wsel, idx)
    loss_commit = psums[0, 0]
    loss = loss_commit * _CC + loss_commit
    quantized_st = quantized.reshape(inputs.shape)
    encoding_indices = idx.reshape(inputs.shape[:-1])
    min_distances = minv2.reshape(inputs.shape[:-1])
    return (quantized_st, loss, encoding_indices, min_distances, loss_commit)
